# baseline (speedup 1.0000x reference)
import functools

import jax
import jax.numpy as jnp
from jax import lax
from jax.experimental import pallas as pl
from jax.experimental.pallas import tpu as pltpu
from jax.experimental.pallas import tpu_sc as plsc

NX, NY = 2048, 2048
N = NX * NY
K = 4
NC, NS = 2, 16
NW = NC * NS
OW = N // NW
CHO = 4096
CH4 = CHO * K
NCH = OW // CHO
LANES = 16
RED_UNROLL = 4


@functools.partial(
    pl.kernel,
    out_type=jax.ShapeDtypeStruct((N,), jnp.float32),
    mesh=plsc.VectorSubcoreMesh(core_axis_name="c", subcore_axis_name="s"),
    compiler_params=pltpu.CompilerParams(needs_layout_passes=False),
    scratch_types=[
        pltpu.VMEM((CH4,), jnp.int32),
        pltpu.VMEM((CH4,), jnp.int32),
        pltpu.VMEM((CH4,), jnp.float32),
        pltpu.VMEM((CH4,), jnp.float32),
        pltpu.VMEM((CH4,), jnp.float32),
        pltpu.VMEM((CH4,), jnp.float32),
        pltpu.VMEM((CHO,), jnp.float32),
        pltpu.VMEM((CHO,), jnp.float32),
        pltpu.SemaphoreType.DMA,
        pltpu.SemaphoreType.DMA,
        pltpu.SemaphoreType.DMA,
        pltpu.SemaphoreType.DMA,
        pltpu.SemaphoreType.DMA,
        pltpu.SemaphoreType.DMA,
    ],
)
def _bilinear_sc(f_hbm, lin_hbm, w_hbm, out_hbm,
                 idx0, idx1, vals0, vals1, w0, w1, outv0, outv1,
                 si0, si1, sg0, sg1, so0, so1):
    wid = lax.axis_index("s") * NC + lax.axis_index("c")
    gbase = wid * OW * K
    obase = wid * OW
    idx_v = (idx0, idx1)
    vals_v = (vals0, vals1)
    w_v = (w0, w1)
    out_v = (outv0, outv1)
    sin = (si0, si1)
    sg = (sg0, sg1)
    so = (so0, so1)

    ii4 = lax.iota(jnp.int32, LANES) * K
    iiuk = [[ii4 + (u * (LANES * K) + k) for k in range(K)]
            for u in range(RED_UNROLL)]

    def fire_in(i):
        b = i % 2
        src = pl.ds(gbase + i * CH4, CH4)
        c_lin = pltpu.async_copy(lin_hbm.at[src], idx_v[b], sin[b])
        c_w = pltpu.async_copy(w_hbm.at[src], w_v[b], sin[b])
        return c_lin, c_w

    def fire_gather(i, pend):
        b = i % 2
        for c in pend[i]:
            c.wait()
        pend[i] = ()
        return pltpu.async_copy(f_hbm.at[idx_v[b]], vals_v[b], sg[b])

    pend = {}
    pend[0] = fire_in(0)
    pend[1] = fire_in(1)
    gathers = {0: fire_gather(0, pend)}
    outs = {}

    for i in range(NCH):
        b = i % 2
        if i + 1 < NCH:
            gathers[i + 1] = fire_gather(i + 1, pend)
        gathers.pop(i).wait()
        if i >= 2:
            outs.pop(i).wait()

        def red(j, _, b=b):
            jbase = j * (LANES * K * RED_UNROLL)
            for u in range(RED_UNROLL):
                acc = None
                for k in range(K):
                    idxk = iiuk[u][k] + jbase
                    vk = plsc.load_gather(vals_v[b], [idxk])
                    wk = plsc.load_gather(w_v[b], [idxk])
                    p = vk * wk
                    acc = p if acc is None else acc + p
                out_v[b][pl.ds(j * (LANES * RED_UNROLL) + u * LANES, LANES)] = acc
            return 0

        lax.fori_loop(0, CHO // (LANES * RED_UNROLL), red, 0)

        outs[i + 2] = pltpu.async_copy(
            out_v[b], out_hbm.at[pl.ds(obase + i * CHO, CHO)], so[b])
        if i + 2 < NCH:
            pend[i + 2] = fire_in(i + 2)

    outs.pop(NCH).wait()
    outs.pop(NCH + 1).wait()


def kernel(f_plane, ix, iy, w, dl):
    nx, ny = f_plane.shape
    lin = (ix.astype(jnp.int32) * ny + iy.astype(jnp.int32)).reshape(-1)
    out = _bilinear_sc(f_plane.reshape(-1), lin, w.reshape(-1))
    return out.reshape(nx, ny)


# speedup vs baseline: 11.0366x; 11.0366x over previous
import functools

import jax
import jax.numpy as jnp
from jax import lax
from jax.experimental import pallas as pl
from jax.experimental.pallas import tpu as pltpu
from jax.experimental.pallas import tpu_sc as plsc

NX, NY = 2048, 2048
N = NX * NY
K = 4
NC, NS = 2, 16
NW = NC * NS
OW = N // NW
CHO = 4096
CH4 = CHO * K
NCH = OW // CHO
LANES = 16
RED_UNROLL = 4


@functools.partial(
    pl.kernel,
    out_type=jax.ShapeDtypeStruct((N,), jnp.float32),
    mesh=plsc.VectorSubcoreMesh(core_axis_name="c", subcore_axis_name="s"),
    scratch_types=[
        pltpu.VMEM((CH4,), jnp.int32),
        pltpu.VMEM((CH4,), jnp.int32),
        pltpu.VMEM((CH4,), jnp.float32),
        pltpu.VMEM((CH4,), jnp.float32),
        pltpu.VMEM((CH4,), jnp.float32),
        pltpu.VMEM((CH4,), jnp.float32),
        pltpu.VMEM((CHO,), jnp.float32),
        pltpu.VMEM((CHO,), jnp.float32),
        pltpu.SemaphoreType.DMA,
        pltpu.SemaphoreType.DMA,
        pltpu.SemaphoreType.DMA,
        pltpu.SemaphoreType.DMA,
        pltpu.SemaphoreType.DMA,
        pltpu.SemaphoreType.DMA,
    ],
)
def _bilinear_sc(f_hbm, lin_hbm, w_hbm, out_hbm,
                 idx0, idx1, vals0, vals1, w0, w1, outv0, outv1,
                 si0, si1, sg0, sg1, so0, so1):
    wid = lax.axis_index("s") * NC + lax.axis_index("c")
    obase = wid * OW
    idx_v = (idx0, idx1)
    vals_v = (vals0, vals1)
    w_v = (w0, w1)
    out_v = (outv0, outv1)
    sin = (si0, si1)
    sg = (sg0, sg1)
    so = (so0, so1)

    def fire_in(i):
        b = i % 2
        copies = []
        for k in range(K):
            src = pl.ds(k * N + obase + i * CHO, CHO)
            dst = pl.ds(k * CHO, CHO)
            copies.append(
                pltpu.async_copy(lin_hbm.at[src], idx_v[b].at[dst], sin[b]))
            copies.append(
                pltpu.async_copy(w_hbm.at[src], w_v[b].at[dst], sin[b]))
        return tuple(copies)

    def fire_gather(i, pend):
        b = i % 2
        for c in pend[i]:
            c.wait()
        pend[i] = ()
        return pltpu.async_copy(f_hbm.at[idx_v[b]], vals_v[b], sg[b])

    pend = {}
    pend[0] = fire_in(0)
    pend[1] = fire_in(1)
    gathers = {0: fire_gather(0, pend)}
    outs = {}

    for i in range(NCH):
        b = i % 2
        if i + 1 < NCH:
            gathers[i + 1] = fire_gather(i + 1, pend)
        gathers.pop(i).wait()
        if i >= 2:
            outs.pop(i).wait()

        def red(j, _, b=b):
            jbase = j * (LANES * RED_UNROLL)
            for u in range(RED_UNROLL):
                acc = None
                for k in range(K):
                    s = pl.ds(k * CHO + u * LANES + jbase, LANES)
                    p = vals_v[b][s] * w_v[b][s]
                    acc = p if acc is None else acc + p
                out_v[b][pl.ds(u * LANES + jbase, LANES)] = acc
            return 0

        lax.fori_loop(0, CHO // (LANES * RED_UNROLL), red, 0)

        outs[i + 2] = pltpu.async_copy(
            out_v[b], out_hbm.at[pl.ds(obase + i * CHO, CHO)], so[b])
        if i + 2 < NCH:
            pend[i + 2] = fire_in(i + 2)

    outs.pop(NCH).wait()
    outs.pop(NCH + 1).wait()


def kernel(f_plane, ix, iy, w, dl):
    nx, ny = f_plane.shape
    lin = ix.astype(jnp.int32) * ny + iy.astype(jnp.int32)
    lin_t = jnp.transpose(lin, (2, 0, 1)).reshape(-1)
    w_t = jnp.transpose(w, (2, 0, 1)).reshape(-1)
    out = _bilinear_sc(f_plane.reshape(-1), lin_t, w_t)
    return out.reshape(nx, ny)
